# baseline (speedup 1.0000x reference)
import functools

import jax
import jax.numpy as jnp
from jax import lax
from jax.experimental import pallas as pl
from jax.experimental.pallas import tpu as pltpu
from jax.experimental.pallas import tpu_sc as plsc

_N = 10000
_NPAD = 10240
_E = 320000
_G = 64
_NC = 2
_NS = 16
_NW = _NC * _NS
_B = 128
_NCHUNK = 80
_EPW = _NCHUNK * _B
_EPAD = _NW * _EPW
_RPT = _NPAD // _NS
_NBUF = 8

_MESH = dict(core_axis_name="c", subcore_axis_name="s")


def _make_deg_kernel():
    @functools.partial(
        pl.kernel,
        out_type=jax.ShapeDtypeStruct((_NC * _NPAD,), jnp.float32),
        mesh=plsc.VectorSubcoreMesh(**_MESH),
        scratch_types=[
            pltpu.VMEM((_NCHUNK, _B), jnp.int32),
            pltpu.VMEM((_B,), jnp.float32),
            pltpu.VMEM((_RPT,), jnp.float32),
            pltpu.VMEM_SHARED((_NPAD,), jnp.float32),
            pltpu.SemaphoreType.DMA,
        ],
    )
    def deg_kernel(dst_hbm, out_hbm, dst_v, ones_v, zb_v, deg_sh, sem):
        c = lax.axis_index("c")
        s = lax.axis_index("s")
        wid = c * _NS + s

        def fill_ones(i, carry):
            ones_v[pl.ds(i * 16, 16)] = jnp.ones((16,), jnp.float32)
            return carry

        lax.fori_loop(0, _B // 16, fill_ones, 0)

        def fill_zero(i, carry):
            zb_v[pl.ds(i * 16, 16)] = jnp.zeros((16,), jnp.float32)
            return carry

        lax.fori_loop(0, _RPT // 16, fill_zero, 0)

        pltpu.sync_copy(dst_hbm.at[wid], dst_v)
        pltpu.sync_copy(zb_v, deg_sh.at[pl.ds(s * _RPT, _RPT)])
        plsc.subcore_barrier()

        def fire8(i, carry):
            descs = [
                pltpu.async_copy(ones_v, deg_sh.at[dst_v.at[i * 8 + b]], sem,
                                 add=True)
                for b in range(8)
            ]
            for d in descs:
                d.wait()
            return carry

        lax.fori_loop(0, _NCHUNK // 8, fire8, 0)
        plsc.subcore_barrier()

        pltpu.sync_copy(deg_sh.at[pl.ds(s * _RPT, _RPT)], zb_v)
        pltpu.sync_copy(zb_v, out_hbm.at[pl.ds(c * _NPAD + s * _RPT, _RPT)])

    return deg_kernel


def _make_agg_kernel(F):
    @functools.partial(
        pl.kernel,
        out_type=jax.ShapeDtypeStruct((_NC * _NPAD, F), jnp.float32),
        mesh=plsc.VectorSubcoreMesh(**_MESH),
        compiler_params=pltpu.CompilerParams(use_tc_tiling_on_sc=False),
        scratch_types=[
            pltpu.VMEM((_NCHUNK, _B), jnp.int32),
            pltpu.VMEM((_NCHUNK, _B), jnp.int32),
            pltpu.VMEM((_NBUF, _B, F), jnp.float32),
            pltpu.VMEM_SHARED((_NPAD, F), jnp.float32),
        ] + [pltpu.SemaphoreType.DMA] * (2 * _NBUF),
    )
    def agg_kernel(y_hbm, src_hbm, dst_hbm, zeros_hbm, out_hbm,
                   src_v, dst_v, rows_v, acc_sh, *sems):
        gsem = sems[:_NBUF]
        ssem = sems[_NBUF:]
        c = lax.axis_index("c")
        s = lax.axis_index("s")
        wid = c * _NS + s

        pltpu.sync_copy(src_hbm.at[wid], src_v)
        pltpu.sync_copy(dst_hbm.at[wid], dst_v)

        pltpu.sync_copy(zeros_hbm.at[pl.ds(0, _B)], rows_v.at[0])
        for t in range(_RPT // _B):
            pltpu.sync_copy(rows_v.at[0], acc_sh.at[pl.ds(s * _RPT + t * _B, _B)])
        plsc.subcore_barrier()

        for b in range(_NBUF):
            pltpu.async_copy(y_hbm.at[src_v.at[b]], rows_v.at[b], gsem[b])

        def step(i, carry):
            for b in range(_NBUF):
                j = i * _NBUF + b
                pltpu.make_async_copy(
                    y_hbm.at[src_v.at[j]], rows_v.at[b], gsem[b]).wait()
                pltpu.async_copy(rows_v.at[b], acc_sh.at[dst_v.at[j]],
                                 ssem[b], add=True)
            for b in range(_NBUF):
                nj = i * _NBUF + b + _NBUF

                @pl.when(nj < _NCHUNK)
                def _():
                    pltpu.make_async_copy(
                        rows_v.at[b], acc_sh.at[dst_v.at[b]],
                        ssem[b]).wait()
                    pltpu.async_copy(
                        y_hbm.at[src_v.at[nj]], rows_v.at[b], gsem[b])
            return carry

        lax.fori_loop(0, _NCHUNK // _NBUF, step, 0)
        for b in range(_NBUF):
            pltpu.make_async_copy(
                rows_v.at[b], acc_sh.at[dst_v.at[b]], ssem[b]).wait()
        plsc.subcore_barrier()

        for t in range(_RPT // _B):
            off = s * _RPT + t * _B
            pltpu.sync_copy(acc_sh.at[pl.ds(off, _B)], rows_v.at[0])
            pltpu.sync_copy(rows_v.at[0], out_hbm.at[pl.ds(c * _NPAD + off, _B)])

    return agg_kernel


_deg_kernel = _make_deg_kernel()
_agg64_kernel = _make_agg_kernel(64)


_RB = 1024


def _tc_scale_matmul(deg2, x_p, w1):
    def body(deg_ref, x_ref, w_ref, dinv_ref, y_ref):
        d = deg_ref[0, :] + deg_ref[1, :] + 1.0
        dinv = lax.rsqrt(d)
        dinv_ref[...] = dinv[:, None]
        y = jnp.dot(x_ref[...], w_ref[...],
                    preferred_element_type=jnp.float32) * dinv[:, None]
        y_ref[0] = y[:, :64]
        y_ref[1] = y[:, 64:]

    return pl.pallas_call(
        body,
        grid=(_NPAD // _RB,),
        in_specs=[
            pl.BlockSpec((_NC, _RB), lambda i: (0, i)),
            pl.BlockSpec((_RB, 128), lambda i: (i, 0)),
            pl.BlockSpec((128, 128), lambda i: (0, 0)),
        ],
        out_specs=[
            pl.BlockSpec((_RB, 1), lambda i: (i, 0)),
            pl.BlockSpec((2, _RB, 64), lambda i: (0, i, 0)),
        ],
        out_shape=[
            jax.ShapeDtypeStruct((_NPAD, 1), jnp.float32),
            jax.ShapeDtypeStruct((2, _NPAD, 64), jnp.float32),
        ],
    )(deg2, x_p, w1)


def _tc_combine(acc_h0, acc_h1, y1h, dinv, b, w):
    def body(a0_ref, a1_ref, y_ref, dinv_ref, b_ref, w_ref, out_ref):
        dinv = dinv_ref[...]
        t0 = (a0_ref[0] + a0_ref[1] + y_ref[0]) * dinv + b_ref[:, :64]
        t1 = (a1_ref[0] + a1_ref[1] + y_ref[1]) * dinv + b_ref[:, 64:]
        t0 = jnp.maximum(t0, 0.0)
        t1 = jnp.maximum(t1, 0.0)
        out_ref[...] = (
            jnp.dot(t0, w_ref[:64], preferred_element_type=jnp.float32)
            + jnp.dot(t1, w_ref[64:], preferred_element_type=jnp.float32)
        ) * dinv

    return pl.pallas_call(
        body,
        grid=(_NPAD // _RB,),
        in_specs=[
            pl.BlockSpec((_NC, _RB, 64), lambda i: (0, i, 0)),
            pl.BlockSpec((_NC, _RB, 64), lambda i: (0, i, 0)),
            pl.BlockSpec((2, _RB, 64), lambda i: (0, i, 0)),
            pl.BlockSpec((_RB, 1), lambda i: (i, 0)),
            pl.BlockSpec((1, 128), lambda i: (0, 0)),
            pl.BlockSpec((128, 64), lambda i: (0, 0)),
        ],
        out_specs=pl.BlockSpec((_RB, 64), lambda i: (i, 0)),
        out_shape=jax.ShapeDtypeStruct((_NPAD, 64), jnp.float32),
    )(acc_h0, acc_h1, y1h, dinv, b, w)


def _tc_final(acc, y, dinv, b, batch_p, wl, bl, F):
    grid = _NPAD // _RB

    def body(acc_ref, y_ref, dinv_ref, b_ref, batch_ref, wl_ref, bl_ref,
             pooled_ref, out_ref):
        i = pl.program_id(0)
        h = (acc_ref[0] + acc_ref[1] + y_ref[...]) * dinv_ref[...] + b_ref[...]
        h = jnp.maximum(h, 0.0)
        gid = lax.broadcasted_iota(jnp.int32, (1, _G), 1)
        onehot = (batch_ref[...] == gid).astype(jnp.float32)
        contrib = lax.dot_general(onehot, h, (((0,), (0,)), ((), ())),
                                  preferred_element_type=jnp.float32)

        @pl.when(i == 0)
        def _():
            pooled_ref[...] = contrib

        @pl.when(i > 0)
        def _():
            pooled_ref[...] += contrib

        @pl.when(i == grid - 1)
        def _():
            out_ref[...] = jnp.dot(pooled_ref[...], wl_ref[...],
                                   preferred_element_type=jnp.float32) + bl_ref[...]

    return pl.pallas_call(
        body,
        grid=(grid,),
        in_specs=[
            pl.BlockSpec((_NC, _RB, F), lambda i: (0, i, 0)),
            pl.BlockSpec((_RB, F), lambda i: (i, 0)),
            pl.BlockSpec((_RB, 1), lambda i: (i, 0)),
            pl.BlockSpec((1, F), lambda i: (0, 0)),
            pl.BlockSpec((_RB, 1), lambda i: (i, 0)),
            pl.BlockSpec((F, 16), lambda i: (0, 0)),
            pl.BlockSpec((1, 16), lambda i: (0, 0)),
        ],
        out_specs=[
            pl.BlockSpec((_G, F), lambda i: (0, 0)),
            pl.BlockSpec((_G, 16), lambda i: (0, 0)),
        ],
        out_shape=[
            jax.ShapeDtypeStruct((_G, F), jnp.float32),
            jax.ShapeDtypeStruct((_G, 16), jnp.float32),
        ],
    )(acc, y, dinv, b, batch_p, wl, bl)


@jax.jit
def kernel(x, edge_index, batch, W1, b1, W2, b2, Wl, bl):
    src = edge_index[0]
    dst = edge_index[1]
    pad_e = _EPAD - _E
    src_p = jnp.concatenate(
        [src, jnp.zeros((pad_e,), src.dtype)]).reshape(_NW, _NCHUNK, _B)
    dst_p = jnp.concatenate(
        [dst, jnp.full((pad_e,), _NPAD - 1, dst.dtype)]).reshape(
            _NW, _NCHUNK, _B)
    x_p = jnp.pad(x, ((0, _NPAD - _N), (0, 0)))
    batch_p = jnp.pad(batch, (0, _NPAD - _N),
                      constant_values=_G).reshape(_NPAD, 1)

    zeros64 = jnp.zeros((_NPAD, 64), jnp.float32)
    deg2 = _deg_kernel(dst_p).reshape(_NC, _NPAD)
    dinv, y1h = _tc_scale_matmul(deg2, x_p, W1)
    acc1h0 = _agg64_kernel(
        y1h[0], src_p, dst_p, zeros64).reshape(_NC, _NPAD, 64)
    acc1h1 = _agg64_kernel(
        y1h[1], src_p, dst_p, zeros64).reshape(_NC, _NPAD, 64)
    y2 = _tc_combine(acc1h0, acc1h1, y1h, dinv, b1.reshape(1, -1), W2)
    acc2 = _agg64_kernel(
        y2, src_p, dst_p, zeros64).reshape(_NC, _NPAD, 64)
    pooled, out = _tc_final(acc2, y2, dinv, b2.reshape(1, -1), batch_p,
                            Wl, bl.reshape(1, -1), 64)
    return (pooled, out)


# speedup vs baseline: 1.2228x; 1.2228x over previous
import functools

import jax
import jax.numpy as jnp
from jax import lax
from jax.experimental import pallas as pl
from jax.experimental.pallas import tpu as pltpu
from jax.experimental.pallas import tpu_sc as plsc

_N = 10000
_NPAD = 10240
_E = 320000
_G = 64
_NC = 2
_NS = 16
_NW = _NC * _NS
_B = 128
_CH0 = 136
_CH1 = 24
_TCH = _NS * (_CH0 + _CH1)
_EPAD = _TCH * _B
_RPT = _NPAD // _NS
_NBUF = 4

_MESH = dict(core_axis_name="c", subcore_axis_name="s")


def _make_deg_kernel():
    @functools.partial(
        pl.kernel,
        out_type=jax.ShapeDtypeStruct((_NC * _NPAD,), jnp.float32),
        mesh=plsc.VectorSubcoreMesh(**_MESH),
        scratch_types=[
            pltpu.VMEM((_CH0, _B), jnp.int32),
            pltpu.VMEM((_B,), jnp.float32),
            pltpu.VMEM((_RPT,), jnp.float32),
            pltpu.VMEM_SHARED((_NPAD,), jnp.float32),
            pltpu.SemaphoreType.DMA,
        ],
    )
    def deg_kernel(dst_hbm, out_hbm, dst_v, ones_v, zb_v, deg_sh, sem):
        c = lax.axis_index("c")
        s = lax.axis_index("s")
        start = jnp.where(c == 0, s * _CH0, _NS * _CH0 + s * _CH1)
        cnt = jnp.where(c == 0, _CH0, _CH1)

        def fill_ones(i, carry):
            ones_v[pl.ds(i * 16, 16)] = jnp.ones((16,), jnp.float32)
            return carry

        lax.fori_loop(0, _B // 16, fill_ones, 0)

        def fill_zero(i, carry):
            zb_v[pl.ds(i * 16, 16)] = jnp.zeros((16,), jnp.float32)
            return carry

        lax.fori_loop(0, _RPT // 16, fill_zero, 0)

        @pl.when(c == 0)
        def _():
            pltpu.sync_copy(dst_hbm.at[pl.ds(start, _CH0)], dst_v)

        @pl.when(c == 1)
        def _():
            pltpu.sync_copy(dst_hbm.at[pl.ds(start, _CH1)],
                            dst_v.at[pl.ds(0, _CH1)])

        pltpu.sync_copy(zb_v, deg_sh.at[pl.ds(s * _RPT, _RPT)])
        plsc.subcore_barrier()

        def fire4(i, carry):
            descs = [
                pltpu.async_copy(ones_v, deg_sh.at[dst_v.at[i * 4 + b]], sem,
                                 add=True)
                for b in range(4)
            ]
            for d in descs:
                d.wait()
            return carry

        lax.fori_loop(0, cnt // 4, fire4, 0)
        plsc.subcore_barrier()

        pltpu.sync_copy(deg_sh.at[pl.ds(s * _RPT, _RPT)], zb_v)
        pltpu.sync_copy(zb_v, out_hbm.at[pl.ds(c * _NPAD + s * _RPT, _RPT)])

    return deg_kernel


def _make_agg_kernel(F):
    @functools.partial(
        pl.kernel,
        out_type=jax.ShapeDtypeStruct((_NC * _NPAD, F), jnp.float32),
        mesh=plsc.VectorSubcoreMesh(**_MESH),
        compiler_params=pltpu.CompilerParams(use_tc_tiling_on_sc=False),
        scratch_types=[
            pltpu.VMEM((_CH0, _B), jnp.int32),
            pltpu.VMEM((_CH0, _B), jnp.int32),
            pltpu.VMEM((_NBUF, _B, F), jnp.float32),
            pltpu.VMEM_SHARED((_NPAD, F), jnp.float32),
        ] + [pltpu.SemaphoreType.DMA] * _NBUF,
    )
    def agg_kernel(y_hbm, src_hbm, dst_hbm, zeros_hbm, out_hbm,
                   src_v, dst_v, rows_v, acc_sh, *sems):
        gsem = sems
        c = lax.axis_index("c")
        s = lax.axis_index("s")
        start = jnp.where(c == 0, s * _CH0, _NS * _CH0 + s * _CH1)
        cnt = jnp.where(c == 0, _CH0, _CH1)

        @pl.when(c == 0)
        def _():
            pltpu.sync_copy(src_hbm.at[pl.ds(start, _CH0)], src_v)
            pltpu.sync_copy(dst_hbm.at[pl.ds(start, _CH0)], dst_v)

        @pl.when(c == 1)
        def _():
            pltpu.sync_copy(src_hbm.at[pl.ds(start, _CH1)],
                            src_v.at[pl.ds(0, _CH1)])
            pltpu.sync_copy(dst_hbm.at[pl.ds(start, _CH1)],
                            dst_v.at[pl.ds(0, _CH1)])

        pltpu.sync_copy(zeros_hbm.at[pl.ds(0, _B)], rows_v.at[0])
        for t in range(_RPT // _B):
            pltpu.sync_copy(rows_v.at[0], acc_sh.at[pl.ds(s * _RPT + t * _B, _B)])
        plsc.subcore_barrier()

        for b in range(_NBUF):
            pltpu.async_copy(y_hbm.at[src_v.at[b]], rows_v.at[b], gsem[b])

        def step(i, carry):
            for b in range(_NBUF):
                j = i * _NBUF + b
                pltpu.make_async_copy(
                    y_hbm.at[src_v.at[j]], rows_v.at[b], gsem[b]).wait()
                pltpu.sync_copy(rows_v.at[b], acc_sh.at[dst_v.at[j]],
                                add=True)
                nj = j + _NBUF

                @pl.when(nj < cnt)
                def _():
                    pltpu.async_copy(
                        y_hbm.at[src_v.at[nj]], rows_v.at[b], gsem[b])
            return carry

        lax.fori_loop(0, cnt // _NBUF, step, 0)
        plsc.subcore_barrier()

        for t in range(_RPT // _B):
            off = s * _RPT + t * _B
            pltpu.sync_copy(acc_sh.at[pl.ds(off, _B)], rows_v.at[0])
            pltpu.sync_copy(rows_v.at[0], out_hbm.at[pl.ds(c * _NPAD + off, _B)])

    return agg_kernel


_deg_kernel = _make_deg_kernel()
_agg64_kernel = _make_agg_kernel(64)


_RB = 1024


def _tc_scale_matmul(deg2, x_p, w1):
    def body(deg_ref, x_ref, w_ref, dinv_ref, y_ref):
        d = deg_ref[0, :] + deg_ref[1, :] + 1.0
        dinv = lax.rsqrt(d)
        dinv_ref[...] = dinv[:, None]
        y = jnp.dot(x_ref[...], w_ref[...],
                    preferred_element_type=jnp.float32) * dinv[:, None]
        y_ref[0] = y[:, :64]
        y_ref[1] = y[:, 64:]

    return pl.pallas_call(
        body,
        grid=(_NPAD // _RB,),
        in_specs=[
            pl.BlockSpec((_NC, _RB), lambda i: (0, i)),
            pl.BlockSpec((_RB, 128), lambda i: (i, 0)),
            pl.BlockSpec((128, 128), lambda i: (0, 0)),
        ],
        out_specs=[
            pl.BlockSpec((_RB, 1), lambda i: (i, 0)),
            pl.BlockSpec((2, _RB, 64), lambda i: (0, i, 0)),
        ],
        out_shape=[
            jax.ShapeDtypeStruct((_NPAD, 1), jnp.float32),
            jax.ShapeDtypeStruct((2, _NPAD, 64), jnp.float32),
        ],
    )(deg2, x_p, w1)


def _tc_combine(acc_h0, acc_h1, y1h, dinv, b, w):
    def body(a0_ref, a1_ref, y_ref, dinv_ref, b_ref, w_ref, out_ref):
        dinv = dinv_ref[...]
        t0 = (a0_ref[0] + a0_ref[1] + y_ref[0]) * dinv + b_ref[:, :64]
        t1 = (a1_ref[0] + a1_ref[1] + y_ref[1]) * dinv + b_ref[:, 64:]
        t0 = jnp.maximum(t0, 0.0)
        t1 = jnp.maximum(t1, 0.0)
        out_ref[...] = (
            jnp.dot(t0, w_ref[:64], preferred_element_type=jnp.float32)
            + jnp.dot(t1, w_ref[64:], preferred_element_type=jnp.float32)
        ) * dinv

    return pl.pallas_call(
        body,
        grid=(_NPAD // _RB,),
        in_specs=[
            pl.BlockSpec((_NC, _RB, 64), lambda i: (0, i, 0)),
            pl.BlockSpec((_NC, _RB, 64), lambda i: (0, i, 0)),
            pl.BlockSpec((2, _RB, 64), lambda i: (0, i, 0)),
            pl.BlockSpec((_RB, 1), lambda i: (i, 0)),
            pl.BlockSpec((1, 128), lambda i: (0, 0)),
            pl.BlockSpec((128, 64), lambda i: (0, 0)),
        ],
        out_specs=pl.BlockSpec((_RB, 64), lambda i: (i, 0)),
        out_shape=jax.ShapeDtypeStruct((_NPAD, 64), jnp.float32),
    )(acc_h0, acc_h1, y1h, dinv, b, w)


def _tc_final(acc, y, dinv, b, batch_p, wl, bl, F):
    grid = _NPAD // _RB

    def body(acc_ref, y_ref, dinv_ref, b_ref, batch_ref, wl_ref, bl_ref,
             pooled_ref, out_ref):
        i = pl.program_id(0)
        h = (acc_ref[0] + acc_ref[1] + y_ref[...]) * dinv_ref[...] + b_ref[...]
        h = jnp.maximum(h, 0.0)
        gid = lax.broadcasted_iota(jnp.int32, (1, _G), 1)
        onehot = (batch_ref[...] == gid).astype(jnp.float32)
        contrib = lax.dot_general(onehot, h, (((0,), (0,)), ((), ())),
                                  preferred_element_type=jnp.float32)

        @pl.when(i == 0)
        def _():
            pooled_ref[...] = contrib

        @pl.when(i > 0)
        def _():
            pooled_ref[...] += contrib

        @pl.when(i == grid - 1)
        def _():
            out_ref[...] = jnp.dot(pooled_ref[...], wl_ref[...],
                                   preferred_element_type=jnp.float32) + bl_ref[...]

    return pl.pallas_call(
        body,
        grid=(grid,),
        in_specs=[
            pl.BlockSpec((_NC, _RB, F), lambda i: (0, i, 0)),
            pl.BlockSpec((_RB, F), lambda i: (i, 0)),
            pl.BlockSpec((_RB, 1), lambda i: (i, 0)),
            pl.BlockSpec((1, F), lambda i: (0, 0)),
            pl.BlockSpec((_RB, 1), lambda i: (i, 0)),
            pl.BlockSpec((F, 16), lambda i: (0, 0)),
            pl.BlockSpec((1, 16), lambda i: (0, 0)),
        ],
        out_specs=[
            pl.BlockSpec((_G, F), lambda i: (0, 0)),
            pl.BlockSpec((_G, 16), lambda i: (0, 0)),
        ],
        out_shape=[
            jax.ShapeDtypeStruct((_G, F), jnp.float32),
            jax.ShapeDtypeStruct((_G, 16), jnp.float32),
        ],
    )(acc, y, dinv, b, batch_p, wl, bl)


@jax.jit
def kernel(x, edge_index, batch, W1, b1, W2, b2, Wl, bl):
    src = edge_index[0]
    dst = edge_index[1]
    pad_e = _EPAD - _E
    src_p = jnp.concatenate(
        [src, jnp.zeros((pad_e,), src.dtype)]).reshape(_TCH, _B)
    dst_p = jnp.concatenate(
        [dst, jnp.full((pad_e,), _NPAD - 1, dst.dtype)]).reshape(_TCH, _B)
    x_p = jnp.pad(x, ((0, _NPAD - _N), (0, 0)))
    batch_p = jnp.pad(batch, (0, _NPAD - _N),
                      constant_values=_G).reshape(_NPAD, 1)

    zeros64 = jnp.zeros((_NPAD, 64), jnp.float32)
    deg2 = _deg_kernel(dst_p).reshape(_NC, _NPAD)
    dinv, y1h = _tc_scale_matmul(deg2, x_p, W1)
    acc1h0 = _agg64_kernel(
        y1h[0], src_p, dst_p, zeros64).reshape(_NC, _NPAD, 64)
    acc1h1 = _agg64_kernel(
        y1h[1], src_p, dst_p, zeros64).reshape(_NC, _NPAD, 64)
    y2 = _tc_combine(acc1h0, acc1h1, y1h, dinv, b1.reshape(1, -1), W2)
    acc2 = _agg64_kernel(
        y2, src_p, dst_p, zeros64).reshape(_NC, _NPAD, 64)
    pooled, out = _tc_final(acc2, y2, dinv, b2.reshape(1, -1), batch_p,
                            Wl, bl.reshape(1, -1), 64)
    return (pooled, out)
